# Initial kernel scaffold; baseline (speedup 1.0000x reference)
import jax
import jax.numpy as jnp
from jax.experimental import pallas as pl


def kernel(batch_embeddings, position_mask, W, b, gamma, beta):
    raise NotImplementedError("write your pallas kernel here")


# speedup vs baseline: 2.2057x; 2.2057x over previous
import functools

import jax
import jax.numpy as jnp
from jax.experimental import pallas as pl

H = 768
EPS = 1e-5
ROWS = 256


def _fused_body(x_ref, m_ref, w_ref, b_ref, g_ref, be_ref, o_ref):
    x = x_ref[...]
    w = w_ref[...]
    sp = jax.lax.dot_general(
        x, w, (((1,), (1,)), ((), ())),
        preferred_element_type=jnp.float32,
    )
    z = x + sp + b_ref[...]
    mean = jnp.mean(z, axis=-1, keepdims=True)
    zc = z - mean
    var = jnp.mean(zc * zc, axis=-1, keepdims=True)
    normed = zc * jax.lax.rsqrt(var + EPS) * g_ref[...] + be_ref[...]
    m = m_ref[...]
    o_ref[...] = jnp.where(m == 1, normed, x)


def kernel(batch_embeddings, position_mask, W, b, gamma, beta):
    L, S, H_ = batch_embeddings.shape
    n = L * S
    nblk = n // ROWS
    x = batch_embeddings.reshape(n, H_)
    m = position_mask.astype(jnp.int32).reshape(n, 1)

    out = pl.pallas_call(
        _fused_body,
        grid=(nblk,),
        in_specs=[
            pl.BlockSpec((ROWS, H_), lambda i: (i, 0)),
            pl.BlockSpec((ROWS, 1), lambda i: (i, 0)),
            pl.BlockSpec((H_, H_), lambda i: (0, 0)),
            pl.BlockSpec((1, H_), lambda i: (0, 0)),
            pl.BlockSpec((1, H_), lambda i: (0, 0)),
            pl.BlockSpec((1, H_), lambda i: (0, 0)),
        ],
        out_specs=pl.BlockSpec((ROWS, H_), lambda i: (i, 0)),
        out_shape=jax.ShapeDtypeStruct((n, H_), jnp.float32),
    )(x, m, W, b.reshape(1, H_), gamma.reshape(1, H_), beta.reshape(1, H_))
    return out.reshape(L, S, H_)
